# Initial kernel scaffold; baseline (speedup 1.0000x reference)
import jax
import jax.numpy as jnp
from jax.experimental import pallas as pl


def kernel(h, edge_index, edge_attr, W1, b1, W2, b2, W3, b3, W4, b4):
    raise NotImplementedError("write your pallas kernel here")


# speedup vs baseline: 2.6669x; 2.6669x over previous
import functools

import jax
import jax.numpy as jnp
from jax import lax
from jax.experimental import pallas as pl
from jax.experimental.pallas import tpu as pltpu
from jax.experimental.pallas import tpu_sc as plsc

N_NODES = 10000
N_EDGES = 320000
D_FEAT = 128
HIDDEN = 256

NC = 2
NS = 16
NW = NC * NS
EPW = N_EDGES // NW
EPS = N_EDGES // NS
KCH = 80
RBLK = 80
NRB = N_NODES // RBLK
HALF = HIDDEN // 2


def _gather_body(ridx_hbm, cidx_hbm, h_hbm, s_hbm, t_hbm,
                 ridx_v, cidx_v, buf_s, buf_t, sem_s, sem_t):
    c = lax.axis_index("c")
    s = lax.axis_index("s")
    wid = s * NC + c
    base = wid * EPW

    def body(i, _):
        off = base + i * KCH
        pltpu.sync_copy(ridx_hbm.at[pl.ds(off, KCH)], ridx_v)
        pltpu.sync_copy(cidx_hbm.at[pl.ds(off, KCH)], cidx_v)
        a = pltpu.async_copy(h_hbm.at[ridx_v], buf_s, sem_s)
        b = pltpu.async_copy(h_hbm.at[cidx_v], buf_t, sem_t)
        a.wait()
        b.wait()
        pltpu.sync_copy(buf_s, s_hbm.at[pl.ds(off, KCH)])
        pltpu.sync_copy(buf_t, t_hbm.at[pl.ds(off, KCH)])
        return 0

    lax.fori_loop(0, EPW // KCH, body, 0)


def _gather_rows(row, col, h):
    mesh = plsc.VectorSubcoreMesh(core_axis_name="c", subcore_axis_name="s",
                                  num_cores=NC, num_subcores=NS)
    f = pl.kernel(
        _gather_body,
        out_type=(jax.ShapeDtypeStruct((N_EDGES, D_FEAT), jnp.float32),
                  jax.ShapeDtypeStruct((N_EDGES, D_FEAT), jnp.float32)),
        mesh=mesh,
        scratch_types=[
            pltpu.VMEM((KCH,), jnp.int32),
            pltpu.VMEM((KCH,), jnp.int32),
            pltpu.VMEM((KCH, D_FEAT), jnp.float32),
            pltpu.VMEM((KCH, D_FEAT), jnp.float32),
            pltpu.SemaphoreType.DMA,
            pltpu.SemaphoreType.DMA,
        ],
    )
    return f(row, col, h)


def _scatter_body(mij_hbm, ridx_hbm, zero_hbm, agg_hbm,
                  acc_sh, idx_v, buf, sem_i, sem_d):
    c = lax.axis_index("c")
    s = lax.axis_index("s")

    def init_body(k, _):
        b = s + NS * k

        @pl.when(b < NRB)
        def _():
            r0 = b * RBLK
            pltpu.sync_copy(zero_hbm.at[pl.ds(r0, RBLK)],
                            acc_sh.at[pl.ds(r0, RBLK)])
        return 0

    lax.fori_loop(0, (NRB + NS - 1) // NS, init_body, 0)
    plsc.subcore_barrier()

    base = s * EPS

    def body(i, _):
        off = base + i * KCH
        pltpu.sync_copy(ridx_hbm.at[pl.ds(off, KCH)], idx_v)
        pltpu.sync_copy(mij_hbm.at[pl.ds(off, KCH), pl.ds(c * HALF, HALF)], buf)
        pltpu.sync_copy(buf, acc_sh.at[idx_v], add=True)
        return 0

    lax.fori_loop(0, EPS // KCH, body, 0)
    plsc.subcore_barrier()

    def out_body(k, _):
        b = s + NS * k

        @pl.when(b < NRB)
        def _():
            r0 = b * RBLK
            pltpu.sync_copy(acc_sh.at[pl.ds(r0, RBLK)],
                            agg_hbm.at[pl.ds(r0, RBLK), pl.ds(c * HALF, HALF)])
        return 0

    lax.fori_loop(0, (NRB + NS - 1) // NS, out_body, 0)


def _segment_sum(mij, row, zero):
    mesh = plsc.VectorSubcoreMesh(core_axis_name="c", subcore_axis_name="s",
                                  num_cores=NC, num_subcores=NS)
    f = pl.kernel(
        _scatter_body,
        out_type=jax.ShapeDtypeStruct((N_NODES, HIDDEN), jnp.float32),
        mesh=mesh,
        scratch_types=[
            pltpu.VMEM_SHARED((N_NODES, HALF), jnp.float32),
            pltpu.VMEM((KCH,), jnp.int32),
            pltpu.VMEM((KCH, HALF), jnp.float32),
            pltpu.SemaphoreType.DMA,
            pltpu.SemaphoreType.DMA,
        ],
    )
    return f(mij, row, zero)


E_BLK = 2000


def _edge_mlp_body(s_ref, t_ref, ea_ref, w12_ref, w1c_ref, b1_ref,
                   w2_ref, b2_ref, out_ref):
    x = jnp.concatenate([s_ref[...], t_ref[...]], axis=1).astype(jnp.bfloat16)
    pre = jnp.dot(x, w12_ref[...], preferred_element_type=jnp.float32)
    pre += jnp.dot(ea_ref[...], w1c_ref[...], preferred_element_type=jnp.float32)
    pre += b1_ref[...]
    t = jax.nn.silu(pre).astype(jnp.bfloat16)
    mij = jnp.dot(t, w2_ref[...], preferred_element_type=jnp.float32)
    out_ref[...] = jax.nn.silu(mij + b2_ref[...])


def _edge_mlp(s_bf, t_bf, ea_bf, w12, w1c, b1, w2, b2):
    grid = (N_EDGES // E_BLK,)
    return pl.pallas_call(
        _edge_mlp_body,
        grid=grid,
        in_specs=[
            pl.BlockSpec((E_BLK, D_FEAT), lambda i: (i, 0)),
            pl.BlockSpec((E_BLK, D_FEAT), lambda i: (i, 0)),
            pl.BlockSpec((E_BLK, 16), lambda i: (i, 0)),
            pl.BlockSpec((HIDDEN, HIDDEN), lambda i: (0, 0)),
            pl.BlockSpec((16, HIDDEN), lambda i: (0, 0)),
            pl.BlockSpec((1, HIDDEN), lambda i: (0, 0)),
            pl.BlockSpec((HIDDEN, HIDDEN), lambda i: (0, 0)),
            pl.BlockSpec((1, HIDDEN), lambda i: (0, 0)),
        ],
        out_specs=pl.BlockSpec((E_BLK, HIDDEN), lambda i: (i, 0)),
        out_shape=jax.ShapeDtypeStruct((N_EDGES, HIDDEN), jnp.float32),
        compiler_params=pltpu.CompilerParams(
            dimension_semantics=("arbitrary",)),
    )(s_bf, t_bf, ea_bf, w12, w1c, b1, w2, b2)


N_BLK = 2000


def _node_mlp_body(h_ref, agg_ref, w3a_ref, w3b_ref, b3_ref, w4_ref, b4_ref,
                   out_ref):
    h = h_ref[...]
    pre = jnp.dot(h.astype(jnp.bfloat16), w3a_ref[...],
                  preferred_element_type=jnp.float32)
    pre += jnp.dot(agg_ref[...].astype(jnp.bfloat16), w3b_ref[...],
                   preferred_element_type=jnp.float32)
    hid = jax.nn.silu(pre + b3_ref[...]).astype(jnp.bfloat16)
    out = jnp.dot(hid, w4_ref[...], preferred_element_type=jnp.float32)
    out_ref[...] = h + out + b4_ref[...]


def _node_mlp(h, agg, w3a, w3b, b3, w4, b4):
    grid = (N_NODES // N_BLK,)
    return pl.pallas_call(
        _node_mlp_body,
        grid=grid,
        in_specs=[
            pl.BlockSpec((N_BLK, D_FEAT), lambda i: (i, 0)),
            pl.BlockSpec((N_BLK, HIDDEN), lambda i: (i, 0)),
            pl.BlockSpec((D_FEAT, HIDDEN), lambda i: (0, 0)),
            pl.BlockSpec((HIDDEN, HIDDEN), lambda i: (0, 0)),
            pl.BlockSpec((1, HIDDEN), lambda i: (0, 0)),
            pl.BlockSpec((HIDDEN, D_FEAT), lambda i: (0, 0)),
            pl.BlockSpec((1, D_FEAT), lambda i: (0, 0)),
        ],
        out_specs=pl.BlockSpec((N_BLK, D_FEAT), lambda i: (i, 0)),
        out_shape=jax.ShapeDtypeStruct((N_NODES, D_FEAT), jnp.float32),
        compiler_params=pltpu.CompilerParams(
            dimension_semantics=("arbitrary",)),
    )(h, agg, w3a, w3b, b3, w4, b4)


def kernel(h, edge_index, edge_attr, W1, b1, W2, b2, W3, b3, W4, b4):
    ei = edge_index.astype(jnp.int32)
    row, col = ei[0], ei[1]

    s_f, t_f = _gather_rows(row, col, h)

    w12 = W1[:2 * D_FEAT].astype(jnp.bfloat16)
    w1c = W1[2 * D_FEAT:].astype(jnp.bfloat16)
    mij = _edge_mlp(s_f, t_f, edge_attr.astype(jnp.bfloat16),
                    w12, w1c, b1.reshape(1, HIDDEN),
                    W2.astype(jnp.bfloat16), b2.reshape(1, HIDDEN))

    zero = jnp.zeros((N_NODES, HALF), jnp.float32)
    agg = _segment_sum(mij, row, zero)

    h_new = _node_mlp(h, agg,
                      W3[:D_FEAT].astype(jnp.bfloat16),
                      W3[D_FEAT:].astype(jnp.bfloat16),
                      b3.reshape(1, HIDDEN),
                      W4.astype(jnp.bfloat16), b4.reshape(1, D_FEAT))
    return (h_new, mij)


# speedup vs baseline: 3.6719x; 1.3768x over previous
import functools

import jax
import jax.numpy as jnp
from jax import lax
from jax.experimental import pallas as pl
from jax.experimental.pallas import tpu as pltpu
from jax.experimental.pallas import tpu_sc as plsc

N_NODES = 10000
N_EDGES = 320000
D_FEAT = 128
HIDDEN = 256

NC = 2
NS = 16
NW = NC * NS
EPW = N_EDGES // NW
EPS = N_EDGES // NS
KCH = 80
RBLK = 80
NRB = N_NODES // RBLK
HALF = HIDDEN // 2


NCH_G = EPW // KCH


def _gather_body(ridx_hbm, cidx_hbm, h_hbm, s_hbm, t_hbm,
                 ridx_v, cidx_v, buf_s0, buf_s1, buf_t0, buf_t1,
                 gs0, gs1, gt0, gt1, ws0, ws1, wt0, wt1):
    c = lax.axis_index("c")
    s = lax.axis_index("s")
    wid = s * NC + c
    base = wid * EPW
    pltpu.sync_copy(ridx_hbm.at[pl.ds(base, EPW)], ridx_v)
    pltpu.sync_copy(cidx_hbm.at[pl.ds(base, EPW)], cidx_v)

    buf_s = (buf_s0, buf_s1)
    buf_t = (buf_t0, buf_t1)
    gs = (gs0, gs1)
    gt = (gt0, gt1)
    ws = (ws0, ws1)
    wt = (wt0, wt1)

    def issue_gather(k, p):
        pltpu.async_copy(h_hbm.at[ridx_v.at[pl.ds(k * KCH, KCH)]],
                         buf_s[p], gs[p])
        pltpu.async_copy(h_hbm.at[cidx_v.at[pl.ds(k * KCH, KCH)]],
                         buf_t[p], gt[p])

    def wait_gather(p):
        pltpu.make_async_copy(h_hbm.at[pl.ds(0, KCH)], buf_s[p], gs[p]).wait()
        pltpu.make_async_copy(h_hbm.at[pl.ds(0, KCH)], buf_t[p], gt[p]).wait()

    def issue_write(k, p):
        off = base + k * KCH
        pltpu.async_copy(buf_s[p], s_hbm.at[pl.ds(off, KCH)], ws[p])
        pltpu.async_copy(buf_t[p], t_hbm.at[pl.ds(off, KCH)], wt[p])

    def wait_write(p):
        pltpu.make_async_copy(buf_s[p], s_hbm.at[pl.ds(0, KCH)], ws[p]).wait()
        pltpu.make_async_copy(buf_t[p], t_hbm.at[pl.ds(0, KCH)], wt[p]).wait()

    issue_gather(0, 0)

    def step(k, p):
        wait_gather(p)

        @pl.when(k > 0)
        def _():
            wait_write(1 - p)

        @pl.when(k < NCH_G - 1)
        def _():
            issue_gather(k + 1, 1 - p)

        issue_write(k, p)

    def body(j, _):
        step(2 * j, 0)
        step(2 * j + 1, 1)
        return 0

    lax.fori_loop(0, NCH_G // 2, body, 0)
    step(NCH_G - 1, (NCH_G - 1) % 2)
    wait_write((NCH_G - 1) % 2)


def _gather_rows(row, col, h):
    mesh = plsc.VectorSubcoreMesh(core_axis_name="c", subcore_axis_name="s",
                                  num_cores=NC, num_subcores=NS)
    f = pl.kernel(
        _gather_body,
        out_type=(jax.ShapeDtypeStruct((N_EDGES, D_FEAT), jnp.float32),
                  jax.ShapeDtypeStruct((N_EDGES, D_FEAT), jnp.float32)),
        mesh=mesh,
        scratch_types=[
            pltpu.VMEM((EPW,), jnp.int32),
            pltpu.VMEM((EPW,), jnp.int32),
            pltpu.VMEM((KCH, D_FEAT), jnp.float32),
            pltpu.VMEM((KCH, D_FEAT), jnp.float32),
            pltpu.VMEM((KCH, D_FEAT), jnp.float32),
            pltpu.VMEM((KCH, D_FEAT), jnp.float32),
        ] + [pltpu.SemaphoreType.DMA] * 8,
    )
    return f(row, col, h)


NCH_S = EPS // KCH


def _scatter_body(mij_hbm, ridx_hbm, zero_hbm, agg_hbm,
                  acc_sh, idx_v0, idx_v1, buf0, buf1,
                  si0, si1, sm0, sm1, sa0, sa1):
    c = lax.axis_index("c")
    s = lax.axis_index("s")

    def init_body(k, _):
        b = s + NS * k

        @pl.when(b < NRB)
        def _():
            r0 = b * RBLK
            pltpu.sync_copy(zero_hbm.at[pl.ds(r0, RBLK)],
                            acc_sh.at[pl.ds(r0, RBLK)])
        return 0

    lax.fori_loop(0, (NRB + NS - 1) // NS, init_body, 0)
    plsc.subcore_barrier()

    base = s * EPS
    idx = (idx_v0, idx_v1)
    buf = (buf0, buf1)
    si = (si0, si1)
    sm = (sm0, sm1)
    sa = (sa0, sa1)

    def issue_fetch(k, p):
        off = base + k * KCH
        pltpu.async_copy(ridx_hbm.at[pl.ds(off, KCH)], idx[p], si[p])
        pltpu.async_copy(mij_hbm.at[pl.ds(off, KCH), pl.ds(c * HALF, HALF)],
                         buf[p], sm[p])

    def wait_fetch(p):
        pltpu.make_async_copy(ridx_hbm.at[pl.ds(0, KCH)], idx[p], si[p]).wait()
        pltpu.make_async_copy(mij_hbm.at[pl.ds(0, KCH), pl.ds(0, HALF)],
                              buf[p], sm[p]).wait()

    def wait_add(p):
        pltpu.make_async_copy(mij_hbm.at[pl.ds(0, KCH), pl.ds(0, HALF)],
                              buf[p], sa[p]).wait()

    issue_fetch(0, 0)

    def step(k, p):
        wait_fetch(p)
        pltpu.async_copy(buf[p], acc_sh.at[idx[p]], sa[p], add=True)

        @pl.when(k > 0)
        def _():
            wait_add(1 - p)

        @pl.when(k < NCH_S - 1)
        def _():
            issue_fetch(k + 1, 1 - p)

    def body(j, _):
        step(2 * j, 0)
        step(2 * j + 1, 1)
        return 0

    lax.fori_loop(0, NCH_S // 2, body, 0)
    wait_add((NCH_S - 1) % 2)
    plsc.subcore_barrier()

    def out_body(k, _):
        b = s + NS * k

        @pl.when(b < NRB)
        def _():
            r0 = b * RBLK
            pltpu.sync_copy(acc_sh.at[pl.ds(r0, RBLK)],
                            agg_hbm.at[pl.ds(r0, RBLK), pl.ds(c * HALF, HALF)])
        return 0

    lax.fori_loop(0, (NRB + NS - 1) // NS, out_body, 0)


def _segment_sum(mij, row, zero):
    mesh = plsc.VectorSubcoreMesh(core_axis_name="c", subcore_axis_name="s",
                                  num_cores=NC, num_subcores=NS)
    f = pl.kernel(
        _scatter_body,
        out_type=jax.ShapeDtypeStruct((N_NODES, HIDDEN), jnp.float32),
        mesh=mesh,
        scratch_types=[
            pltpu.VMEM_SHARED((N_NODES, HALF), jnp.float32),
            pltpu.VMEM((KCH,), jnp.int32),
            pltpu.VMEM((KCH,), jnp.int32),
            pltpu.VMEM((KCH, HALF), jnp.float32),
            pltpu.VMEM((KCH, HALF), jnp.float32),
        ] + [pltpu.SemaphoreType.DMA] * 6,
    )
    return f(mij, row, zero)


E_BLK = 2000


def _edge_mlp_body(s_ref, t_ref, ea_ref, w12_ref, w1c_ref, b1_ref,
                   w2_ref, b2_ref, out_ref):
    x = jnp.concatenate([s_ref[...], t_ref[...]], axis=1).astype(jnp.bfloat16)
    pre = jnp.dot(x, w12_ref[...], preferred_element_type=jnp.float32)
    pre += jnp.dot(ea_ref[...], w1c_ref[...], preferred_element_type=jnp.float32)
    pre += b1_ref[...]
    t = jax.nn.silu(pre).astype(jnp.bfloat16)
    mij = jnp.dot(t, w2_ref[...], preferred_element_type=jnp.float32)
    out_ref[...] = jax.nn.silu(mij + b2_ref[...])


def _edge_mlp(s_bf, t_bf, ea_bf, w12, w1c, b1, w2, b2):
    grid = (N_EDGES // E_BLK,)
    return pl.pallas_call(
        _edge_mlp_body,
        grid=grid,
        in_specs=[
            pl.BlockSpec((E_BLK, D_FEAT), lambda i: (i, 0)),
            pl.BlockSpec((E_BLK, D_FEAT), lambda i: (i, 0)),
            pl.BlockSpec((E_BLK, 16), lambda i: (i, 0)),
            pl.BlockSpec((HIDDEN, HIDDEN), lambda i: (0, 0)),
            pl.BlockSpec((16, HIDDEN), lambda i: (0, 0)),
            pl.BlockSpec((1, HIDDEN), lambda i: (0, 0)),
            pl.BlockSpec((HIDDEN, HIDDEN), lambda i: (0, 0)),
            pl.BlockSpec((1, HIDDEN), lambda i: (0, 0)),
        ],
        out_specs=pl.BlockSpec((E_BLK, HIDDEN), lambda i: (i, 0)),
        out_shape=jax.ShapeDtypeStruct((N_EDGES, HIDDEN), jnp.float32),
        compiler_params=pltpu.CompilerParams(
            dimension_semantics=("arbitrary",)),
    )(s_bf, t_bf, ea_bf, w12, w1c, b1, w2, b2)


N_BLK = 2000


def _node_mlp_body(h_ref, agg_ref, w3a_ref, w3b_ref, b3_ref, w4_ref, b4_ref,
                   out_ref):
    h = h_ref[...]
    pre = jnp.dot(h.astype(jnp.bfloat16), w3a_ref[...],
                  preferred_element_type=jnp.float32)
    pre += jnp.dot(agg_ref[...].astype(jnp.bfloat16), w3b_ref[...],
                   preferred_element_type=jnp.float32)
    hid = jax.nn.silu(pre + b3_ref[...]).astype(jnp.bfloat16)
    out = jnp.dot(hid, w4_ref[...], preferred_element_type=jnp.float32)
    out_ref[...] = h + out + b4_ref[...]


def _node_mlp(h, agg, w3a, w3b, b3, w4, b4):
    grid = (N_NODES // N_BLK,)
    return pl.pallas_call(
        _node_mlp_body,
        grid=grid,
        in_specs=[
            pl.BlockSpec((N_BLK, D_FEAT), lambda i: (i, 0)),
            pl.BlockSpec((N_BLK, HIDDEN), lambda i: (i, 0)),
            pl.BlockSpec((D_FEAT, HIDDEN), lambda i: (0, 0)),
            pl.BlockSpec((HIDDEN, HIDDEN), lambda i: (0, 0)),
            pl.BlockSpec((1, HIDDEN), lambda i: (0, 0)),
            pl.BlockSpec((HIDDEN, D_FEAT), lambda i: (0, 0)),
            pl.BlockSpec((1, D_FEAT), lambda i: (0, 0)),
        ],
        out_specs=pl.BlockSpec((N_BLK, D_FEAT), lambda i: (i, 0)),
        out_shape=jax.ShapeDtypeStruct((N_NODES, D_FEAT), jnp.float32),
        compiler_params=pltpu.CompilerParams(
            dimension_semantics=("arbitrary",)),
    )(h, agg, w3a, w3b, b3, w4, b4)


def kernel(h, edge_index, edge_attr, W1, b1, W2, b2, W3, b3, W4, b4):
    ei = edge_index.astype(jnp.int32)
    row, col = ei[0], ei[1]

    s_f, t_f = _gather_rows(row, col, h)

    w12 = W1[:2 * D_FEAT].astype(jnp.bfloat16)
    w1c = W1[2 * D_FEAT:].astype(jnp.bfloat16)
    mij = _edge_mlp(s_f, t_f, edge_attr.astype(jnp.bfloat16),
                    w12, w1c, b1.reshape(1, HIDDEN),
                    W2.astype(jnp.bfloat16), b2.reshape(1, HIDDEN))

    zero = jnp.zeros((N_NODES, HALF), jnp.float32)
    agg = _segment_sum(mij, row, zero)

    h_new = _node_mlp(h, agg,
                      W3[:D_FEAT].astype(jnp.bfloat16),
                      W3[D_FEAT:].astype(jnp.bfloat16),
                      b3.reshape(1, HIDDEN),
                      W4.astype(jnp.bfloat16), b4.reshape(1, D_FEAT))
    return (h_new, mij)
